# Initial kernel scaffold; baseline (speedup 1.0000x reference)
import jax
import jax.numpy as jnp
from jax.experimental import pallas as pl


def kernel(x_indices, x_offsets, tables, W1, b1, W2, b2, W3, b3):
    raise NotImplementedError("write your pallas kernel here")


# speedup vs baseline: 68.4970x; 68.4970x over previous
import functools

import jax
import jax.numpy as jnp
from jax import lax
from jax.experimental import pallas as pl
from jax.experimental.pallas import tpu as pltpu
from jax.experimental.pallas import tpu_sc as plsc

_N_SLOTS = 26
_VOCAB = 100000
_EMB = 32
_B = 4096
_L = 20480
_NC = 2
_NS = 16
_SLOTS_PER_CORE = _N_SLOTS // _NC
_ROUND1 = 7
_W = _L // _NS
_CH = 128
_NCHUNK = _W // _CH
_LG2B = 12


def _sc_body(idx_hbm, off_hbm, tab_hbm, out_hbm,
             idx_v, rows_v, off_v, ind_v, seg_v, acc_sh, gsem, ssem):
    c = lax.axis_index("c")
    t = lax.axis_index("s")
    p0 = t * _W

    zeros16f = jnp.zeros((16,), jnp.float32)
    zeros16i = jnp.zeros((16,), jnp.int32)
    e0 = (lax.iota(jnp.int32, 16) == 0).astype(jnp.int32)

    def _zrow(i, carry):
        rows_v[i, pl.ds(0, 16)] = zeros16f
        rows_v[i, pl.ds(16, 16)] = zeros16f
        return carry
    lax.fori_loop(0, 256, _zrow, 0)

    def _searchsorted_left(v):
        def _it(_, lohi):
            lo, hi = lohi
            mid = (lo + hi) // 2
            ov = off_v[pl.ds(mid, 16)][0]
            pred = ov < v
            return (jnp.where(pred, mid + 1, lo), jnp.where(pred, hi, mid))
        lo, _ = lax.fori_loop(0, _LG2B, _it,
                              (jnp.int32(0), jnp.int32(_B)))
        return lo

    def _make_slot(rb):
      def _slot(sl, carry):
        slot = c * _SLOTS_PER_CORE + rb + sl

        pltpu.sync_copy(off_hbm.at[pl.ds(pl.multiple_of(slot * _B, 8), _B)],
                        off_v.at[pl.ds(0, _B)])
        pltpu.sync_copy(
            idx_hbm.at[pl.ds(pl.multiple_of(slot * _L + p0, 8), _W)],
            idx_v)

        ibase = slot * _VOCAB
        for k in range(_W // 16):
            v = idx_v[pl.ds(k * 16, 16)]
            idx_v[pl.ds(k * 16, 16)] = v + ibase

        gds = []
        for j in range(_NCHUNK):
            gds.append(pltpu.async_copy(
                tab_hbm.at[idx_v.at[pl.ds(j * _CH, _CH)]],
                rows_v.at[pl.ds(j * _CH, _CH)], gsem))
        for d in gds:
            d.wait()

        base = _searchsorted_left(p0)
        jend = _searchsorted_left(p0 + _W)

        for k in range(_W // 16):
            ind_v[pl.ds(k * 16, 16)] = zeros16i

        def _bag(j, carry):
            q = off_v[pl.ds(j, 16)][0] - p0
            vec = ind_v[pl.ds(q, 16)]
            ind_v[pl.ds(q, 16)] = vec + e0
            return carry
        lax.fori_loop(base, jend, _bag, 0)

        carry_s = sl * _B + base - 1
        for k in range(_W // 16):
            vec = ind_v[pl.ds(k * 16, 16)]
            seg16 = plsc.cumsum(vec) + carry_s
            seg_v[k // (_CH // 16), pl.ds((k % (_CH // 16)) * 16, 16)] = seg16
            carry_s = carry_s + jnp.sum(vec)

        for j in range(_NCHUNK):
            pltpu.sync_copy(rows_v.at[pl.ds(j * _CH, _CH)],
                            acc_sh.at[seg_v.at[j]], add=True)
        return carry
      return _slot

    for rb, nsl in ((0, _ROUND1), (_ROUND1, _SLOTS_PER_CORE - _ROUND1)):
        def _zacc(sl, carry):
            pltpu.sync_copy(rows_v.at[pl.ds(0, 256)],
                            acc_sh.at[pl.ds(sl * _B + t * 256, 256)])
            return carry
        lax.fori_loop(0, nsl, _zacc, 0)
        plsc.subcore_barrier()

        lax.fori_loop(0, nsl, _make_slot(rb), 0)
        plsc.subcore_barrier()

        def _rd(sl, carry, rb=rb):
            slot = c * _SLOTS_PER_CORE + rb + sl
            pltpu.sync_copy(acc_sh.at[pl.ds(sl * _B + t * 256, 256)],
                            out_hbm.at[pl.ds(slot * _B + t * 256, 256)])
            return carry
        lax.fori_loop(0, nsl, _rd, 0)


@jax.jit
def _sc_bag_sums(idx3, off_flat, tab2):
    mesh = plsc.VectorSubcoreMesh(core_axis_name="c", subcore_axis_name="s")
    f = functools.partial(
        pl.kernel,
        out_type=jax.ShapeDtypeStruct((_N_SLOTS * _B, _EMB), jnp.float32),
        mesh=mesh,
        scratch_types=[
            pltpu.VMEM((_W,), jnp.int32),
            pltpu.VMEM((_W, _EMB), jnp.float32),
            pltpu.VMEM((_B + 16,), jnp.int32),
            pltpu.VMEM((_W + 16,), jnp.int32),
            pltpu.VMEM((_NCHUNK, _CH), jnp.int32),
            pltpu.VMEM_SHARED((_ROUND1 * _B, _EMB), jnp.float32),
            pltpu.SemaphoreType.DMA,
            pltpu.SemaphoreType.DMA,
        ],
        compiler_params=pltpu.CompilerParams(needs_layout_passes=False,
                                             use_tc_tiling_on_sc=False),
    )(_sc_body)
    return f(idx3, off_flat, tab2)


def _mlp_body(sums_ref, off_ref, ends_ref, w1_ref, b1_ref, w2_ref, b2_ref,
              w3_ref, b3_ref, out_ref):
    cnt = (ends_ref[...] - off_ref[...]).astype(jnp.float32)
    scale = jnp.where(cnt > 0.0, 1.0 / jnp.maximum(cnt, 1.0), 0.0)
    acc = jnp.zeros((off_ref.shape[1], 256), jnp.float32)
    acc = acc + b1_ref[...][None, :]
    for s in range(_N_SLOTS):
        xs = sums_ref[s] * scale[s][:, None]
        acc = acc + jnp.dot(xs, w1_ref[s],
                            preferred_element_type=jnp.float32,
                            precision=lax.Precision.HIGHEST)
    h1 = jnp.maximum(acc, 0.0)
    h2 = jnp.maximum(
        jnp.dot(h1, w2_ref[...], preferred_element_type=jnp.float32,
                precision=lax.Precision.HIGHEST) + b2_ref[...][None, :], 0.0)
    z = jnp.dot(h2, w3_ref[...], preferred_element_type=jnp.float32,
                precision=lax.Precision.HIGHEST) + b3_ref[...][None, :]
    out_ref[...] = jax.nn.sigmoid(z)


@jax.jit
def _tc_mlp(sums3, off2, ends2, w1r, b1, w2, b2, w3, b3):
    tb = 512
    grid = (_B // tb,)
    return pl.pallas_call(
        _mlp_body,
        grid=grid,
        in_specs=[
            pl.BlockSpec((_N_SLOTS, tb, _EMB), lambda i: (0, i, 0)),
            pl.BlockSpec((_N_SLOTS, tb), lambda i: (0, i)),
            pl.BlockSpec((_N_SLOTS, tb), lambda i: (0, i)),
            pl.BlockSpec((_N_SLOTS, _EMB, 256), lambda i: (0, 0, 0)),
            pl.BlockSpec((256,), lambda i: (0,)),
            pl.BlockSpec((256, 128), lambda i: (0, 0)),
            pl.BlockSpec((128,), lambda i: (0,)),
            pl.BlockSpec((128, 1), lambda i: (0, 0)),
            pl.BlockSpec((1,), lambda i: (0,)),
        ],
        out_specs=pl.BlockSpec((tb, 1), lambda i: (i, 0)),
        out_shape=jax.ShapeDtypeStruct((_B, 1), jnp.float32),
    )(sums3, off2, ends2, w1r, b1, w2, b2, w3, b3)


def kernel(x_indices, x_offsets, tables, W1, b1, W2, b2, W3, b3):
    x_indices = x_indices.astype(jnp.int32)
    x_offsets = x_offsets.astype(jnp.int32)
    idx3 = x_indices.reshape(_N_SLOTS * _L)
    off_flat = x_offsets.reshape(_N_SLOTS * _B)
    tab2 = tables.reshape(_N_SLOTS * _VOCAB, _EMB)

    sums = _sc_bag_sums(idx3, off_flat, tab2)
    sums3 = sums.reshape(_N_SLOTS, _B, _EMB)

    ends2 = jnp.concatenate(
        [x_offsets[:, 1:], jnp.full((_N_SLOTS, 1), _L, jnp.int32)], axis=1)
    w1r = W1.reshape(_N_SLOTS, _EMB, 256)
    return _tc_mlp(sums3, x_offsets, ends2, w1r, b1, W2, b2, W3, b3)


# speedup vs baseline: 70.0930x; 1.0233x over previous
import functools

import jax
import jax.numpy as jnp
from jax import lax
from jax.experimental import pallas as pl
from jax.experimental.pallas import tpu as pltpu
from jax.experimental.pallas import tpu_sc as plsc

_N_SLOTS = 26
_VOCAB = 100000
_EMB = 32
_B = 4096
_L = 20480
_NC = 2
_NS = 16
_SLOTS_PER_CORE = _N_SLOTS // _NC
_ROUND1 = 7
_W = _L // _NS
_CH = 128
_NCHUNK = _W // _CH
_LG2B = 12


def _sc_body(idx_hbm, off_hbm, tab_hbm, out_hbm,
             idx_v, rows_v, off_v, ind_v, seg_v, acc_sh, gsem, ssem):
    c = lax.axis_index("c")
    t = lax.axis_index("s")
    p0 = t * _W

    zeros16f = jnp.zeros((16,), jnp.float32)
    zeros16i = jnp.zeros((16,), jnp.int32)
    e0 = (lax.iota(jnp.int32, 16) == 0).astype(jnp.int32)

    def _zrow(i, carry):
        rows_v[i, pl.ds(0, 16)] = zeros16f
        rows_v[i, pl.ds(16, 16)] = zeros16f
        return carry
    lax.fori_loop(0, 256, _zrow, 0)

    def _searchsorted_left(v):
        def _it(_, lohi):
            lo, hi = lohi
            mid = (lo + hi) // 2
            ov = off_v[pl.ds(mid, 16)][0]
            pred = ov < v
            return (jnp.where(pred, mid + 1, lo), jnp.where(pred, hi, mid))
        lo, _ = lax.fori_loop(0, _LG2B, _it,
                              (jnp.int32(0), jnp.int32(_B)))
        return lo

    def _make_slot(rb):
      def _slot(sl, carry):
        slot = c * _SLOTS_PER_CORE + rb + sl

        pltpu.sync_copy(off_hbm.at[pl.ds(pl.multiple_of(slot * _B, 8), _B)],
                        off_v.at[pl.ds(0, _B)])
        pltpu.sync_copy(
            idx_hbm.at[pl.ds(pl.multiple_of(slot * _L + p0, 8), _W)],
            idx_v)

        ibase = slot * _VOCAB
        for k in range(_W // 16):
            v = idx_v[pl.ds(k * 16, 16)]
            idx_v[pl.ds(k * 16, 16)] = v + ibase

        gds = []
        for j in range(_NCHUNK):
            gds.append(pltpu.async_copy(
                tab_hbm.at[idx_v.at[pl.ds(j * _CH, _CH)]],
                rows_v.at[pl.ds(j * _CH, _CH)], gsem))

        base = _searchsorted_left(p0)
        jend = _searchsorted_left(p0 + _W)

        for k in range(_W // 16):
            ind_v[pl.ds(k * 16, 16)] = zeros16i

        def _bag(j, carry):
            q = off_v[pl.ds(j, 16)][0] - p0
            vec = ind_v[pl.ds(q, 16)]
            ind_v[pl.ds(q, 16)] = vec + e0
            return carry
        lax.fori_loop(base, jend, _bag, 0)

        carry_s = sl * _B + base - 1
        for k in range(_W // 16):
            vec = ind_v[pl.ds(k * 16, 16)]
            seg16 = plsc.cumsum(vec) + carry_s
            seg_v[k // (_CH // 16), pl.ds((k % (_CH // 16)) * 16, 16)] = seg16
            carry_s = carry_s + jnp.sum(vec)

        for d in gds:
            d.wait()
        sds = []
        for j in range(_NCHUNK):
            sds.append(pltpu.async_copy(rows_v.at[pl.ds(j * _CH, _CH)],
                                        acc_sh.at[seg_v.at[j]], ssem,
                                        add=True))
        for d in sds:
            d.wait()
        return carry
      return _slot

    for rb, nsl in ((0, _ROUND1), (_ROUND1, _SLOTS_PER_CORE - _ROUND1)):
        def _zacc(sl, carry):
            pltpu.sync_copy(rows_v.at[pl.ds(0, 256)],
                            acc_sh.at[pl.ds(sl * _B + t * 256, 256)])
            return carry
        lax.fori_loop(0, nsl, _zacc, 0)
        plsc.subcore_barrier()

        lax.fori_loop(0, nsl, _make_slot(rb), 0)
        plsc.subcore_barrier()

        def _rd(sl, carry, rb=rb):
            slot = c * _SLOTS_PER_CORE + rb + sl
            pltpu.sync_copy(acc_sh.at[pl.ds(sl * _B + t * 256, 256)],
                            out_hbm.at[pl.ds(slot * _B + t * 256, 256)])
            return carry
        lax.fori_loop(0, nsl, _rd, 0)


@jax.jit
def _sc_bag_sums(idx3, off_flat, tab2):
    mesh = plsc.VectorSubcoreMesh(core_axis_name="c", subcore_axis_name="s")
    f = functools.partial(
        pl.kernel,
        out_type=jax.ShapeDtypeStruct((_N_SLOTS * _B, _EMB), jnp.float32),
        mesh=mesh,
        scratch_types=[
            pltpu.VMEM((_W,), jnp.int32),
            pltpu.VMEM((_W, _EMB), jnp.float32),
            pltpu.VMEM((_B + 16,), jnp.int32),
            pltpu.VMEM((_W + 16,), jnp.int32),
            pltpu.VMEM((_NCHUNK, _CH), jnp.int32),
            pltpu.VMEM_SHARED((_ROUND1 * _B, _EMB), jnp.float32),
            pltpu.SemaphoreType.DMA,
            pltpu.SemaphoreType.DMA,
        ],
        compiler_params=pltpu.CompilerParams(needs_layout_passes=False,
                                             use_tc_tiling_on_sc=False),
    )(_sc_body)
    return f(idx3, off_flat, tab2)


def _mlp_body(sums_ref, off_ref, ends_ref, w1_ref, b1_ref, w2_ref, b2_ref,
              w3_ref, b3_ref, out_ref):
    cnt = (ends_ref[...] - off_ref[...]).astype(jnp.float32)
    scale = jnp.where(cnt > 0.0, 1.0 / jnp.maximum(cnt, 1.0), 0.0)
    acc = jnp.zeros((off_ref.shape[1], 256), jnp.float32)
    acc = acc + b1_ref[...][None, :]
    for s in range(_N_SLOTS):
        xs = sums_ref[s] * scale[s][:, None]
        acc = acc + jnp.dot(xs, w1_ref[s],
                            preferred_element_type=jnp.float32,
                            precision=lax.Precision.HIGHEST)
    h1 = jnp.maximum(acc, 0.0)
    h2 = jnp.maximum(
        jnp.dot(h1, w2_ref[...], preferred_element_type=jnp.float32,
                precision=lax.Precision.HIGHEST) + b2_ref[...][None, :], 0.0)
    z = jnp.dot(h2, w3_ref[...], preferred_element_type=jnp.float32,
                precision=lax.Precision.HIGHEST) + b3_ref[...][None, :]
    out_ref[...] = jax.nn.sigmoid(z)


@jax.jit
def _tc_mlp(sums3, off2, ends2, w1r, b1, w2, b2, w3, b3):
    tb = 512
    grid = (_B // tb,)
    return pl.pallas_call(
        _mlp_body,
        grid=grid,
        in_specs=[
            pl.BlockSpec((_N_SLOTS, tb, _EMB), lambda i: (0, i, 0)),
            pl.BlockSpec((_N_SLOTS, tb), lambda i: (0, i)),
            pl.BlockSpec((_N_SLOTS, tb), lambda i: (0, i)),
            pl.BlockSpec((_N_SLOTS, _EMB, 256), lambda i: (0, 0, 0)),
            pl.BlockSpec((256,), lambda i: (0,)),
            pl.BlockSpec((256, 128), lambda i: (0, 0)),
            pl.BlockSpec((128,), lambda i: (0,)),
            pl.BlockSpec((128, 1), lambda i: (0, 0)),
            pl.BlockSpec((1,), lambda i: (0,)),
        ],
        out_specs=pl.BlockSpec((tb, 1), lambda i: (i, 0)),
        out_shape=jax.ShapeDtypeStruct((_B, 1), jnp.float32),
    )(sums3, off2, ends2, w1r, b1, w2, b2, w3, b3)


def kernel(x_indices, x_offsets, tables, W1, b1, W2, b2, W3, b3):
    x_indices = x_indices.astype(jnp.int32)
    x_offsets = x_offsets.astype(jnp.int32)
    idx3 = x_indices.reshape(_N_SLOTS * _L)
    off_flat = x_offsets.reshape(_N_SLOTS * _B)
    tab2 = tables.reshape(_N_SLOTS * _VOCAB, _EMB)

    sums = _sc_bag_sums(idx3, off_flat, tab2)
    sums3 = sums.reshape(_N_SLOTS, _B, _EMB)

    ends2 = jnp.concatenate(
        [x_offsets[:, 1:], jnp.full((_N_SLOTS, 1), _L, jnp.int32)], axis=1)
    w1r = W1.reshape(_N_SLOTS, _EMB, 256)
    return _tc_mlp(sums3, x_offsets, ends2, w1r, b1, W2, b2, W3, b3)


# speedup vs baseline: 74.1511x; 1.0579x over previous
import functools

import jax
import jax.numpy as jnp
from jax import lax
from jax.experimental import pallas as pl
from jax.experimental.pallas import tpu as pltpu
from jax.experimental.pallas import tpu_sc as plsc

_N_SLOTS = 26
_VOCAB = 100000
_EMB = 32
_B = 4096
_L = 20480
_NC = 2
_NS = 16
_SLOTS_PER_CORE = _N_SLOTS // _NC
_ROUND1 = 7
_W = _L // _NS
_CH = 128
_NCHUNK = _W // _CH
_LG2B = 12


def _sc_body(idx_hbm, off_hbm, tab_hbm, out_hbm,
             idx_v, rows_v, off_v, ind_v, seg_v, acc_sh, gsem, ssem):
    c = lax.axis_index("c")
    t = lax.axis_index("s")
    p0 = t * _W

    zeros16f = jnp.zeros((16,), jnp.float32)
    zeros16i = jnp.zeros((16,), jnp.int32)
    e0 = (lax.iota(jnp.int32, 16) == 0).astype(jnp.int32)

    def _zrow(i, carry):
        rows_v[i, pl.ds(0, 16)] = zeros16f
        rows_v[i, pl.ds(16, 16)] = zeros16f
        return carry
    lax.fori_loop(0, 256, _zrow, 0)

    def _searchsorted_left(v):
        def _it(_, lohi):
            lo, hi = lohi
            mid = (lo + hi) // 2
            ov = off_v[pl.ds(mid, 16)][0]
            pred = ov < v
            return (jnp.where(pred, mid + 1, lo), jnp.where(pred, hi, mid))
        lo, _ = lax.fori_loop(0, _LG2B, _it,
                              (jnp.int32(0), jnp.int32(_B)))
        return lo

    def _make_slot(rb):
      def _slot(sl, carry):
        slot = c * _SLOTS_PER_CORE + rb + sl

        pltpu.sync_copy(off_hbm.at[pl.ds(pl.multiple_of(slot * _B, 8), _B)],
                        off_v.at[pl.ds(0, _B)])
        pltpu.sync_copy(
            idx_hbm.at[pl.ds(pl.multiple_of(slot * _L + p0, 8), _W)],
            idx_v)

        ibase = slot * _VOCAB
        for k in range(_W // 16):
            v = idx_v[pl.ds(k * 16, 16)]
            idx_v[pl.ds(k * 16, 16)] = v + ibase

        gds = []
        for j in range(_NCHUNK):
            gds.append(pltpu.async_copy(
                tab_hbm.at[idx_v.at[pl.ds(j * _CH, _CH)]],
                rows_v.at[pl.ds(j * _CH, _CH)], gsem))

        base = _searchsorted_left(p0)
        jend = _searchsorted_left(p0 + _W)

        for k in range(_W // 16):
            ind_v[pl.ds(k * 16, 16)] = zeros16i

        def _bag(j, carry):
            q = off_v[pl.ds(j, 16)][0] - p0
            vec = ind_v[pl.ds(q, 16)]
            ind_v[pl.ds(q, 16)] = vec + e0
            return carry
        lax.fori_loop(base, jend, _bag, 0)

        carry_s = sl * _B + base - 1
        for k in range(_W // 16):
            vec = ind_v[pl.ds(k * 16, 16)]
            seg16 = plsc.cumsum(vec) + carry_s
            seg_v[k // (_CH // 16), pl.ds((k % (_CH // 16)) * 16, 16)] = seg16
            carry_s = carry_s + jnp.sum(vec)

        for d in gds:
            d.wait()
        sds = []
        for j in range(_NCHUNK):
            sds.append(pltpu.async_copy(rows_v.at[pl.ds(j * _CH, _CH)],
                                        acc_sh.at[seg_v.at[j]], ssem,
                                        add=True))
        for d in sds:
            d.wait()
        return carry
      return _slot

    for rb, nsl in ((0, _ROUND1), (_ROUND1, _SLOTS_PER_CORE - _ROUND1)):
        def _zacc(sl, carry):
            pltpu.sync_copy(rows_v.at[pl.ds(0, 256)],
                            acc_sh.at[pl.ds(sl * _B + t * 256, 256)])
            return carry
        lax.fori_loop(0, nsl, _zacc, 0)
        plsc.subcore_barrier()

        lax.fori_loop(0, nsl, _make_slot(rb), 0)
        plsc.subcore_barrier()

        def _rd(sl, carry, rb=rb):
            slot = c * _SLOTS_PER_CORE + rb + sl
            pltpu.sync_copy(acc_sh.at[pl.ds(sl * _B + t * 256, 256)],
                            out_hbm.at[pl.ds(slot * _B + t * 256, 256)])
            return carry
        lax.fori_loop(0, nsl, _rd, 0)


@jax.jit
def _sc_bag_sums(idx3, off_flat, tab2):
    mesh = plsc.VectorSubcoreMesh(core_axis_name="c", subcore_axis_name="s")
    f = functools.partial(
        pl.kernel,
        out_type=jax.ShapeDtypeStruct((_N_SLOTS * _B, _EMB), jnp.float32),
        mesh=mesh,
        scratch_types=[
            pltpu.VMEM((_W,), jnp.int32),
            pltpu.VMEM((_W, _EMB), jnp.float32),
            pltpu.VMEM((_B + 16,), jnp.int32),
            pltpu.VMEM((_W + 16,), jnp.int32),
            pltpu.VMEM((_NCHUNK, _CH), jnp.int32),
            pltpu.VMEM_SHARED((_ROUND1 * _B, _EMB), jnp.float32),
            pltpu.SemaphoreType.DMA,
            pltpu.SemaphoreType.DMA,
        ],
        compiler_params=pltpu.CompilerParams(needs_layout_passes=False,
                                             use_tc_tiling_on_sc=False),
    )(_sc_body)
    return f(idx3, off_flat, tab2)


def _mlp_body(sums_ref, off_ref, ends_ref, w1_ref, b1_ref, w2_ref, b2_ref,
              w3_ref, b3_ref, out_ref):
    cnt = (ends_ref[...] - off_ref[...]).astype(jnp.float32)
    scale = jnp.where(cnt > 0.0, 1.0 / jnp.maximum(cnt, 1.0), 0.0)
    acc = jnp.zeros((off_ref.shape[1], 256), jnp.float32)
    acc = acc + b1_ref[...][None, :]
    for s in range(_N_SLOTS):
        xs = sums_ref[s] * scale[s][:, None]
        acc = acc + jnp.dot(xs, w1_ref[s],
                            preferred_element_type=jnp.float32,
                            precision=lax.Precision.DEFAULT)
    h1 = jnp.maximum(acc, 0.0)
    h2 = jnp.maximum(
        jnp.dot(h1, w2_ref[...], preferred_element_type=jnp.float32,
                precision=lax.Precision.DEFAULT) + b2_ref[...][None, :], 0.0)
    z = jnp.dot(h2, w3_ref[...], preferred_element_type=jnp.float32,
                precision=lax.Precision.DEFAULT) + b3_ref[...][None, :]
    out_ref[...] = jax.nn.sigmoid(z)


@jax.jit
def _tc_mlp(sums3, off2, ends2, w1r, b1, w2, b2, w3, b3):
    tb = 512
    grid = (_B // tb,)
    return pl.pallas_call(
        _mlp_body,
        grid=grid,
        in_specs=[
            pl.BlockSpec((_N_SLOTS, tb, _EMB), lambda i: (0, i, 0)),
            pl.BlockSpec((_N_SLOTS, tb), lambda i: (0, i)),
            pl.BlockSpec((_N_SLOTS, tb), lambda i: (0, i)),
            pl.BlockSpec((_N_SLOTS, _EMB, 256), lambda i: (0, 0, 0)),
            pl.BlockSpec((256,), lambda i: (0,)),
            pl.BlockSpec((256, 128), lambda i: (0, 0)),
            pl.BlockSpec((128,), lambda i: (0,)),
            pl.BlockSpec((128, 1), lambda i: (0, 0)),
            pl.BlockSpec((1,), lambda i: (0,)),
        ],
        out_specs=pl.BlockSpec((tb, 1), lambda i: (i, 0)),
        out_shape=jax.ShapeDtypeStruct((_B, 1), jnp.float32),
    )(sums3, off2, ends2, w1r, b1, w2, b2, w3, b3)


def kernel(x_indices, x_offsets, tables, W1, b1, W2, b2, W3, b3):
    x_indices = x_indices.astype(jnp.int32)
    x_offsets = x_offsets.astype(jnp.int32)
    idx3 = x_indices.reshape(_N_SLOTS * _L)
    off_flat = x_offsets.reshape(_N_SLOTS * _B)
    tab2 = tables.reshape(_N_SLOTS * _VOCAB, _EMB)

    sums = _sc_bag_sums(idx3, off_flat, tab2)
    sums3 = sums.reshape(_N_SLOTS, _B, _EMB)

    ends2 = jnp.concatenate(
        [x_offsets[:, 1:], jnp.full((_N_SLOTS, 1), _L, jnp.int32)], axis=1)
    w1r = W1.reshape(_N_SLOTS, _EMB, 256)
    return _tc_mlp(sums3, x_offsets, ends2, w1r, b1, W2, b2, W3, b3)


# speedup vs baseline: 75.5940x; 1.0195x over previous
import functools

import jax
import jax.numpy as jnp
from jax import lax
from jax.experimental import pallas as pl
from jax.experimental.pallas import tpu as pltpu
from jax.experimental.pallas import tpu_sc as plsc

_N_SLOTS = 26
_VOCAB = 100000
_EMB = 32
_B = 4096
_L = 20480
_NC = 2
_NS = 16
_SLOTS_PER_CORE = _N_SLOTS // _NC
_ROUND1 = 7
_W = _L // _NS
_CH = 128
_NCHUNK = _W // _CH
_LG2B = 12


def _sc_body(idx_hbm, off_hbm, tab_hbm, out_hbm,
             idx_v, rows_v, off_v, ind_v, seg_v, acc_sh, gsem, ssem):
    c = lax.axis_index("c")
    t = lax.axis_index("s")
    p0 = t * _W

    zeros16f = jnp.zeros((16,), jnp.float32)
    zeros16i = jnp.zeros((16,), jnp.int32)
    e0 = (lax.iota(jnp.int32, 16) == 0).astype(jnp.int32)

    def _zrow(i, carry):
        rows_v[i, pl.ds(0, 16)] = zeros16f
        rows_v[i, pl.ds(16, 16)] = zeros16f
        return carry
    lax.fori_loop(0, 256, _zrow, 0)

    def _searchsorted_left(v):
        def _it(_, lohi):
            lo, hi = lohi
            mid = (lo + hi) // 2
            ov = off_v[pl.ds(mid, 16)][0]
            pred = ov < v
            return (jnp.where(pred, mid + 1, lo), jnp.where(pred, hi, mid))
        lo, _ = lax.fori_loop(0, _LG2B, _it,
                              (jnp.int32(0), jnp.int32(_B)))
        return lo

    def _make_slot(rb):
      def _slot(sl, carry):
        slot = c * _SLOTS_PER_CORE + rb + sl

        pltpu.sync_copy(off_hbm.at[pl.ds(pl.multiple_of(slot * _B, 8), _B)],
                        off_v.at[pl.ds(0, _B)])
        pltpu.sync_copy(
            idx_hbm.at[pl.ds(pl.multiple_of(slot * _L + p0, 8), _W)],
            idx_v)

        ibase = slot * _VOCAB
        for k in range(_W // 16):
            v = idx_v[pl.ds(k * 16, 16)]
            idx_v[pl.ds(k * 16, 16)] = v + ibase

        gds = []
        for j in range(_NCHUNK):
            gds.append(pltpu.async_copy(
                tab_hbm.at[idx_v.at[pl.ds(j * _CH, _CH)]],
                rows_v.at[pl.ds(j * _CH, _CH)], gsem))

        base = _searchsorted_left(p0)
        jend = _searchsorted_left(p0 + _W)

        for k in range(_W // 16):
            ind_v[pl.ds(k * 16, 16)] = zeros16i

        def _bag(j, carry):
            q = off_v[pl.ds(j, 16)][0] - p0
            vec = ind_v[pl.ds(q, 16)]
            ind_v[pl.ds(q, 16)] = vec + e0
            return carry
        lax.fori_loop(base, jend, _bag, 0)

        carry_s = sl * _B + base - 1
        for k in range(_W // 16):
            vec = ind_v[pl.ds(k * 16, 16)]
            seg16 = plsc.cumsum(vec) + carry_s
            seg_v[k // (_CH // 16), pl.ds((k % (_CH // 16)) * 16, 16)] = seg16
            carry_s = carry_s + jnp.sum(vec)

        for d in gds:
            d.wait()
        sds = []
        for j in range(_NCHUNK):
            sds.append(pltpu.async_copy(rows_v.at[pl.ds(j * _CH, _CH)],
                                        acc_sh.at[seg_v.at[j]], ssem,
                                        add=True))
        for d in sds:
            d.wait()
        return carry
      return _slot

    for rb, nsl in ((0, _ROUND1), (_ROUND1, _SLOTS_PER_CORE - _ROUND1)):
        def _zacc(sl, carry):
            pltpu.sync_copy(rows_v.at[pl.ds(0, 256)],
                            acc_sh.at[pl.ds(sl * _B + t * 256, 256)])
            return carry
        lax.fori_loop(0, nsl, _zacc, 0)
        plsc.subcore_barrier()

        lax.fori_loop(0, nsl, _make_slot(rb), 0)
        plsc.subcore_barrier()

        def _rd(sl, carry, rb=rb):
            slot = c * _SLOTS_PER_CORE + rb + sl
            pltpu.sync_copy(
                acc_sh.at[pl.ds(sl * _B + t * 256, 256)],
                out_hbm.at[pl.ds(slot * _B + t * 256, 256), pl.ds(0, _EMB)])
            return carry
        lax.fori_loop(0, nsl, _rd, 0)


@jax.jit
def _sc_bag_sums(idx3, off_flat, tab2):
    mesh = plsc.VectorSubcoreMesh(core_axis_name="c", subcore_axis_name="s")
    f = functools.partial(
        pl.kernel,
        out_type=jax.ShapeDtypeStruct((_N_SLOTS * _B, 128), jnp.float32),
        mesh=mesh,
        scratch_types=[
            pltpu.VMEM((_W,), jnp.int32),
            pltpu.VMEM((_W, _EMB), jnp.float32),
            pltpu.VMEM((_B + 16,), jnp.int32),
            pltpu.VMEM((_W + 16,), jnp.int32),
            pltpu.VMEM((_NCHUNK, _CH), jnp.int32),
            pltpu.VMEM_SHARED((_ROUND1 * _B, _EMB), jnp.float32),
            pltpu.SemaphoreType.DMA,
            pltpu.SemaphoreType.DMA,
        ],
        compiler_params=pltpu.CompilerParams(needs_layout_passes=False,
                                             use_tc_tiling_on_sc=False),
    )(_sc_body)
    return f(idx3, off_flat, tab2)


def _mlp_body(sums_ref, off_ref, ends_ref, w1_ref, b1_ref, w2_ref, b2_ref,
              w3_ref, b3_ref, out_ref):
    cnt = (ends_ref[...] - off_ref[...]).astype(jnp.float32)
    scale = jnp.where(cnt > 0.0, 1.0 / jnp.maximum(cnt, 1.0), 0.0)
    tb = off_ref.shape[1]
    acc = jnp.zeros((tb, 256), jnp.float32)
    acc = acc + b1_ref[...][None, :]
    for s in range(_N_SLOTS):
        xs = sums_ref[s][:, 0:_EMB] * scale[s][:, None]
        acc = acc + jnp.dot(xs, w1_ref[s],
                            preferred_element_type=jnp.float32,
                            precision=lax.Precision.DEFAULT)
    h1 = jnp.maximum(acc, 0.0)
    h2 = jnp.maximum(
        jnp.dot(h1, w2_ref[...], preferred_element_type=jnp.float32,
                precision=lax.Precision.DEFAULT) + b2_ref[...][None, :], 0.0)
    z = jnp.dot(h2, w3_ref[...], preferred_element_type=jnp.float32,
                precision=lax.Precision.DEFAULT) + b3_ref[...][None, :]
    out_ref[...] = jax.nn.sigmoid(z)


@jax.jit
def _tc_mlp(sums3, off2, ends2, w1r, b1, w2, b2, w3, b3):
    tb = 512
    grid = (_B // tb,)
    return pl.pallas_call(
        _mlp_body,
        grid=grid,
        in_specs=[
            pl.BlockSpec((_N_SLOTS, tb, 128), lambda i: (0, i, 0)),
            pl.BlockSpec((_N_SLOTS, tb), lambda i: (0, i)),
            pl.BlockSpec((_N_SLOTS, tb), lambda i: (0, i)),
            pl.BlockSpec((_N_SLOTS, _EMB, 256), lambda i: (0, 0, 0)),
            pl.BlockSpec((256,), lambda i: (0,)),
            pl.BlockSpec((256, 128), lambda i: (0, 0)),
            pl.BlockSpec((128,), lambda i: (0,)),
            pl.BlockSpec((128, 1), lambda i: (0, 0)),
            pl.BlockSpec((1,), lambda i: (0,)),
        ],
        out_specs=pl.BlockSpec((tb, 1), lambda i: (i, 0)),
        out_shape=jax.ShapeDtypeStruct((_B, 1), jnp.float32),
    )(sums3, off2, ends2, w1r, b1, w2, b2, w3, b3)


def kernel(x_indices, x_offsets, tables, W1, b1, W2, b2, W3, b3):
    x_indices = x_indices.astype(jnp.int32)
    x_offsets = x_offsets.astype(jnp.int32)
    idx3 = x_indices.reshape(_N_SLOTS * _L)
    off_flat = x_offsets.reshape(_N_SLOTS * _B)
    tab2 = tables.reshape(_N_SLOTS * _VOCAB, _EMB)

    sums = _sc_bag_sums(idx3, off_flat, tab2)
    sums3 = sums.reshape(_N_SLOTS, _B, 128)

    ends2 = jnp.concatenate(
        [x_offsets[:, 1:], jnp.full((_N_SLOTS, 1), _L, jnp.int32)], axis=1)
    w1r = W1.reshape(_N_SLOTS, _EMB, 256)
    return _tc_mlp(sums3, x_offsets, ends2, w1r, b1, W2, b2, W3, b3)
